# baseline (speedup 1.0000x reference)
import functools

import jax
import jax.numpy as jnp
from jax import lax
from jax.experimental import pallas as pl
from jax.experimental.pallas import tpu as pltpu
from jax.experimental.pallas import tpu_sc as plsc

_D = 64
_NC = 2
_NS = 16
_NW = _NC * _NS
_CHUNK = 128
_NBUF = 8
_LEAD = 4


@functools.lru_cache(maxsize=None)
def _build(total: int, per_w: int, nchunk: int):
    mesh = plsc.VectorSubcoreMesh(core_axis_name="c", subcore_axis_name="s")
    nouter = nchunk // _NBUF

    @functools.partial(
        pl.kernel,
        mesh=mesh,
        compiler_params=pltpu.CompilerParams(
            use_tc_tiling_on_sc=False, needs_layout_passes=False
        ),
        out_type=jax.ShapeDtypeStruct((total, _D), jnp.float32),
        scratch_types=[
            pltpu.VMEM((nchunk, _CHUNK), jnp.int32),
            pltpu.VMEM((_NBUF, _CHUNK, _D), jnp.float32),
        ]
        + [pltpu.SemaphoreType.DMA] * (2 * _NBUF),
    )
    def k(idx_hbm, tab_hbm, out_hbm, idx_v, g, *sems):
        gsem = sems[:_NBUF]
        ssem = sems[_NBUF:]
        wid = lax.axis_index("s") * _NC + lax.axis_index("c")
        base = wid * per_w
        pltpu.sync_copy(idx_hbm.at[wid], idx_v)

        def gissue(c, b):
            pltpu.async_copy(tab_hbm.at[idx_v.at[c]], g.at[b], gsem[b])

        def gwait(c, b):
            pltpu.make_async_copy(
                tab_hbm.at[idx_v.at[c]], g.at[b], gsem[b]
            ).wait()

        def out_slice(c):
            return out_hbm.at[pl.ds(base + c * _CHUNK, _CHUNK)]

        def sissue(c, b):
            pltpu.async_copy(g.at[b], out_slice(c), ssem[b])

        def swait(c, b):
            pltpu.make_async_copy(g.at[b], out_slice(c), ssem[b]).wait()

        for t in range(_LEAD):
            gissue(t, t)
        for b in range(_NBUF):
            cl = b + _LEAD
            bl = cl % _NBUF
            if b >= _NBUF - _LEAD:
                swait(cl - _NBUF, bl)
            gissue(cl, bl)
            gwait(b, b)
            sissue(b, b)

        def outer(j, carry):
            for b in range(_NBUF):
                c = j * _NBUF + b
                cl = c + _LEAD
                bl = (b + _LEAD) % _NBUF
                swait(cl - _NBUF, bl)
                gissue(cl, bl)
                gwait(c, b)
                sissue(c, b)
            return carry

        lax.fori_loop(1, nouter - 1, outer, 0)

        for b in range(_NBUF):
            c = (nouter - 1) * _NBUF + b
            cl = c + _LEAD
            if cl < nchunk:
                bl = cl % _NBUF
                swait(cl - _NBUF, bl)
                gissue(cl, bl)
            gwait(c, b)
            sissue(c, b)
        for b in range(_NBUF):
            swait((nouter - 1) * _NBUF + b, b)

    return k


def kernel(x, table):
    b, h = x.shape
    n, d = table.shape
    total = b * h
    per_w = total // _NW
    nchunk = per_w // _CHUNK
    assert per_w * _NW == total and nchunk * _CHUNK == per_w and d == _D
    assert nchunk % _NBUF == 0 and nchunk // _NBUF >= 2
    x_resh = x.reshape(_NW, nchunk, _CHUNK)
    tab_f32 = table.astype(jnp.float32)
    out = _build(total, per_w, nchunk)(x_resh, tab_f32)
    out_flat = out.reshape(total * _D) * jnp.float32(8.0)
    return out_flat.astype(jnp.bfloat16).reshape(b, h, _D)


# speedup vs baseline: 1.0005x; 1.0005x over previous
import functools

import jax
import jax.numpy as jnp
from jax import lax
from jax.experimental import pallas as pl
from jax.experimental.pallas import tpu as pltpu
from jax.experimental.pallas import tpu_sc as plsc

_D = 64
_NC = 2
_NS = 16
_NW = _NC * _NS
_CHUNK = 128
_NBUF = 10
_LEAD = 6


@functools.lru_cache(maxsize=None)
def _build(total: int, per_w: int, nchunk: int):
    mesh = plsc.VectorSubcoreMesh(core_axis_name="c", subcore_axis_name="s")
    nouter = nchunk // _NBUF

    @functools.partial(
        pl.kernel,
        mesh=mesh,
        compiler_params=pltpu.CompilerParams(
            use_tc_tiling_on_sc=False, needs_layout_passes=False
        ),
        out_type=jax.ShapeDtypeStruct((total, _D), jnp.float32),
        scratch_types=[
            pltpu.VMEM((nchunk, _CHUNK), jnp.int32),
            pltpu.VMEM((_NBUF, _CHUNK, _D), jnp.float32),
        ]
        + [pltpu.SemaphoreType.DMA] * (2 * _NBUF),
    )
    def k(idx_hbm, tab_hbm, out_hbm, idx_v, g, *sems):
        gsem = sems[:_NBUF]
        ssem = sems[_NBUF:]
        wid = lax.axis_index("s") * _NC + lax.axis_index("c")
        base = wid * per_w
        pltpu.sync_copy(idx_hbm.at[wid], idx_v)

        def gissue(c, b):
            pltpu.async_copy(tab_hbm.at[idx_v.at[c]], g.at[b], gsem[b])

        def gwait(c, b):
            pltpu.make_async_copy(
                tab_hbm.at[idx_v.at[c]], g.at[b], gsem[b]
            ).wait()

        def out_slice(c):
            return out_hbm.at[pl.ds(base + c * _CHUNK, _CHUNK)]

        def sissue(c, b):
            pltpu.async_copy(g.at[b], out_slice(c), ssem[b])

        def swait(c, b):
            pltpu.make_async_copy(g.at[b], out_slice(c), ssem[b]).wait()

        for t in range(_LEAD):
            gissue(t, t)
        for b in range(_NBUF):
            cl = b + _LEAD
            bl = cl % _NBUF
            if b >= _NBUF - _LEAD:
                swait(cl - _NBUF, bl)
            gissue(cl, bl)
            gwait(b, b)
            sissue(b, b)

        def outer(j, carry):
            for b in range(_NBUF):
                c = j * _NBUF + b
                cl = c + _LEAD
                bl = (b + _LEAD) % _NBUF
                swait(cl - _NBUF, bl)
                gissue(cl, bl)
                gwait(c, b)
                sissue(c, b)
            return carry

        lax.fori_loop(1, nouter - 1, outer, 0)

        for b in range(_NBUF):
            c = (nouter - 1) * _NBUF + b
            cl = c + _LEAD
            if cl < nchunk:
                bl = cl % _NBUF
                swait(cl - _NBUF, bl)
                gissue(cl, bl)
            gwait(c, b)
            sissue(c, b)
        for b in range(_NBUF):
            swait((nouter - 1) * _NBUF + b, b)

    return k


def kernel(x, table):
    b, h = x.shape
    n, d = table.shape
    total = b * h
    per_w = total // _NW
    nchunk = per_w // _CHUNK
    assert per_w * _NW == total and nchunk * _CHUNK == per_w and d == _D
    assert nchunk % _NBUF == 0 and nchunk // _NBUF >= 2
    x_resh = x.reshape(_NW, nchunk, _CHUNK)
    tab_f32 = table.astype(jnp.float32)
    out = _build(total, per_w, nchunk)(x_resh, tab_f32)
    out_flat = out.reshape(total * _D) * jnp.float32(8.0)
    return out_flat.astype(jnp.bfloat16).reshape(b, h, _D)
